# baseline (speedup 1.0000x reference)
import functools

import jax
import jax.numpy as jnp
from jax import lax
from jax.experimental import pallas as pl
from jax.experimental.pallas import tpu as pltpu
from jax.experimental.pallas import tpu_sc as plsc

N = 50000
E = 800000
G = 256
DIN = 64
F0, F1, F2 = 128, 256, 512
GH = 256

NC, NS = 2, 16
EB = 128
NBLK = E // EB
CW = 128
NP = 50688
HSEG = NP // NS
EPAD = 2816
NBLKP = (E + EPAD) // EB

BN = 2000


def _make_deg_kernel():
  bp = NBLKP // (NC * NS)

  @functools.partial(
      pl.kernel,
      out_type=jax.ShapeDtypeStruct((NC, NP), jnp.float32),
      mesh=plsc.VectorSubcoreMesh(core_axis_name="c", subcore_axis_name="s"),
      scratch_types=[
          pltpu.VMEM((NP,), jnp.float32),
          pltpu.VMEM((EB,), jnp.int32),
          pltpu.VMEM((HSEG,), jnp.float32),
          pltpu.VMEM((HSEG,), jnp.float32),
          pltpu.VMEM_SHARED((NS, NP), jnp.float32),
      ],
      compiler_params=pltpu.CompilerParams(
          use_tc_tiling_on_sc=False, needs_layout_passes=False),
  )
  def deg_kernel(edges_hbm, out_hbm, hist, didx, acc, tmp, shared):
    c = lax.axis_index("c")
    s = lax.axis_index("s")
    wid = c * NS + s
    zeros16 = jnp.zeros((16,), jnp.float32)
    ones16 = jnp.ones((16,), jnp.float32)

    def zero_hist(i, carry):
      hist[pl.ds(i * 16, 16)] = zeros16
      return carry
    lax.fori_loop(0, NP // 16, zero_hist, 0)

    base = wid * bp

    def blk_body(b, carry):
      off = (base + b) * EB
      pltpu.sync_copy(edges_hbm.at[1, pl.ds(off, EB)], didx)
      def lane_body(j, carry2):
        idx = didx[pl.ds(j * 16, 16)]
        plsc.addupdate_scatter(hist, [idx], ones16)
        return carry2
      lax.fori_loop(0, EB // 16, lane_body, 0)
      return carry
    lax.fori_loop(0, bp, blk_body, 0)

    pltpu.sync_copy(hist, shared.at[s])
    plsc.subcore_barrier()

    def zero_acc(i, carry):
      acc[pl.ds(i * 16, 16)] = zeros16
      return carry
    lax.fori_loop(0, HSEG // 16, zero_acc, 0)
    seg0 = s * HSEG
    for t in range(NS):
      pltpu.sync_copy(shared.at[t, pl.ds(seg0, HSEG)], tmp)
      def radd(i, carry):
        acc[pl.ds(i * 16, 16)] = acc[pl.ds(i * 16, 16)] + tmp[pl.ds(i * 16, 16)]
        return carry
      lax.fori_loop(0, HSEG // 16, radd, 0)
    pltpu.sync_copy(acc, out_hbm.at[c, pl.ds(seg0, HSEG)])

  return deg_kernel


def _make_agg_kernel():
  fin = 128
  NSL = 2
  RNP = 8448
  ZB = 176
  SPS = RNP // NS
  npass = NP // RNP // NC
  bp = NBLKP // NS

  @functools.partial(
      pl.kernel,
      out_type=jax.ShapeDtypeStruct((NP, fin), jnp.float32),
      mesh=plsc.VectorSubcoreMesh(core_axis_name="c", subcore_axis_name="s"),
      scratch_types=[
          pltpu.VMEM((ZB, fin), jnp.float32),
          [pltpu.VMEM((2, EB), jnp.int32)] * NSL,
          [pltpu.VMEM((EB,), jnp.int32)] * NSL,
          [pltpu.VMEM((EB,), jnp.int32)] * NSL,
          [pltpu.VMEM((EB, fin), jnp.float32)] * NSL,
          pltpu.VMEM_SHARED((RNP, fin), jnp.float32),
          [pltpu.SemaphoreType.DMA] * NSL,
          [pltpu.SemaphoreType.DMA] * NSL,
          [pltpu.SemaphoreType.DMA] * NSL,
      ],
  )
  def agg_kernel(hs_hbm, eblk_hbm, out_hbm, zbuf, ev, gi2, si2, rw2,
                 accum, isem, gsem, ssem):
    c = lax.axis_index("c")
    s = lax.axis_index("s")
    zeros16 = jnp.zeros((16,), jnp.float32)

    def zb_body(i, carry):
      for q in range(fin // 16):
        zbuf[i, pl.ds(q * 16, 16)] = zeros16
      return carry
    lax.fori_loop(0, ZB, zb_body, 0)

    bbase = s * bp

    def issue_idx(b, sl):
      bc = jnp.minimum(b, bp - 1)
      pltpu.async_copy(eblk_hbm.at[bbase + bc], ev[sl], isem[sl])

    def wait_idx(sl):
      pltpu.make_async_copy(eblk_hbm.at[0], ev[sl], isem[sl]).wait()

    for pi in range(npass):
      nbase = (c * npass + pi) * RNP

      for z in range(SPS // ZB):
        pltpu.sync_copy(zbuf, accum.at[pl.ds(s * SPS + z * ZB, ZB)])
      plsc.subcore_barrier()

      def compute_masks(sl, nb):
        def lane_body(j, carry2):
          sv = ev[sl][0, pl.ds(j * 16, 16)]
          dv = ev[sl][1, pl.ds(j * 16, 16)]
          inr = (dv >= nb) & (dv < nb + RNP)
          gi2[sl][pl.ds(j * 16, 16)] = jnp.where(inr, sv, -1)
          si2[sl][pl.ds(j * 16, 16)] = jnp.where(inr, dv - nb, -1)
          return carry2
        lax.fori_loop(0, EB // 16, lane_body, 0)

      def start_gather(sl):
        pltpu.async_copy(hs_hbm.at[plsc.Indices(gi2[sl], ignored_value=-1)],
                         rw2[sl], gsem[sl])

      def wait_gather(sl):
        pltpu.make_async_copy(
            hs_hbm.at[plsc.Indices(gi2[sl], ignored_value=-1)], rw2[sl],
            gsem[sl]).wait()

      def start_scatter(sl):
        pltpu.async_copy(rw2[sl],
                         accum.at[plsc.Indices(si2[sl], ignored_value=-1)],
                         ssem[sl], add=True)

      def wait_scatter(sl):
        pltpu.make_async_copy(rw2[sl],
                              accum.at[plsc.Indices(si2[sl],
                                                    ignored_value=-1)],
                              ssem[sl]).wait()

      for sl in range(NSL):
        issue_idx(sl, sl)

      def grp_body(g, carry):
        b0 = NSL * g
        for sl in range(NSL):
          wait_idx(sl)

          @pl.when(g > 0)
          def _():
            wait_scatter(sl)
          compute_masks(sl, nbase)
          start_gather(sl)
          issue_idx(b0 + NSL + sl, sl)
        for sl in range(NSL):
          wait_gather(sl)
          start_scatter(sl)
        return carry
      lax.fori_loop(0, bp // NSL, grp_body, 0)

      for sl in range(NSL):
        wait_scatter(sl)
        wait_idx(sl)
      plsc.subcore_barrier()

      for z in range(SPS // ZB):
        r0 = s * SPS + z * ZB
        pltpu.sync_copy(accum.at[pl.ds(r0, ZB)],
                        out_hbm.at[pl.ds(nbase + r0, ZB)])
      plsc.subcore_barrier()

  return agg_kernel


_deg_call = _make_deg_kernel()
_agg128 = _make_agg_kernel()


def _pre0_tc(deg2t, x, W0):
  def body(deg_ref, x_ref, w_ref, us_ref, dinv_ref):
    d = deg_ref[:, 0] + deg_ref[:, 1] + 1.0
    dv = lax.rsqrt(d)[:, None]
    dinv_ref[...] = dv
    u = jnp.dot(x_ref[...], w_ref[...], preferred_element_type=jnp.float32)
    us_ref[...] = u * dv

  return pl.pallas_call(
      body,
      grid=(N // BN,),
      in_specs=[
          pl.BlockSpec((BN, 2), lambda i: (i, 0)),
          pl.BlockSpec((BN, DIN), lambda i: (i, 0)),
          pl.BlockSpec((DIN, F0), lambda i: (0, 0)),
      ],
      out_specs=[
          pl.BlockSpec((BN, F0), lambda i: (i, 0)),
          pl.BlockSpec((BN, 1), lambda i: (i, 0)),
      ],
      out_shape=[
          jax.ShapeDtypeStruct((N, F0), jnp.float32),
          jax.ShapeDtypeStruct((N, 1), jnp.float32),
      ],
  )(deg2t, x, W0)


def _mid0_tc(S0, us0, dinv, b0):
  def body(S_ref, us_ref, dinv_ref, b_ref, out_ref):
    dv = dinv_ref[...]
    h = jnp.maximum((S_ref[...] + us_ref[...]) * dv + b_ref[...], 0.0)
    out_ref[...] = h * dv

  return pl.pallas_call(
      body,
      grid=(N // BN,),
      in_specs=[
          pl.BlockSpec((BN, F0), lambda i: (i, 0)),
          pl.BlockSpec((BN, F0), lambda i: (i, 0)),
          pl.BlockSpec((BN, 1), lambda i: (i, 0)),
          pl.BlockSpec((1, F0), lambda i: (0, 0)),
      ],
      out_specs=pl.BlockSpec((BN, F0), lambda i: (i, 0)),
      out_shape=jax.ShapeDtypeStruct((N, F0), jnp.float32),
  )(S0, us0, dinv, b0)


def _gcn_tc(S_list, hs_list, dinv, W, b, scale_out):
  fin, fout = W.shape
  nin = len(S_list)
  assert nin * 128 == fin and len(hs_list) == nin
  nout = fout // 128 if scale_out else 1

  def body(*refs):
    S_refs = refs[:nin]
    hs_refs = refs[nin:2 * nin]
    dinv_ref, W_ref, b_ref = refs[2 * nin:2 * nin + 3]
    out_refs = refs[2 * nin + 3:]
    dv = dinv_ref[...]
    if nin == 1:
      t = (S_refs[0][...] + hs_refs[0][...]) * dv
    else:
      t = jnp.concatenate(
          [S_refs[q][...] + hs_refs[q][...] for q in range(nin)], axis=1) * dv
    t = jnp.dot(t, W_ref[...], preferred_element_type=jnp.float32) + b_ref[...]
    h = jnp.maximum(t, 0.0)
    if scale_out:
      h = h * dv
      for q in range(nout):
        out_refs[q][...] = h[:, q * 128:(q + 1) * 128]
    else:
      out_refs[0][...] = h

  piece = lambda: pl.BlockSpec((BN, 128), lambda i: (i, 0))
  out_w = 128 if scale_out else fout
  return pl.pallas_call(
      body,
      grid=(N // BN,),
      in_specs=(
          [piece() for _ in range(2 * nin)] + [
              pl.BlockSpec((BN, 1), lambda i: (i, 0)),
              pl.BlockSpec((fin, fout), lambda i: (0, 0)),
              pl.BlockSpec((1, fout), lambda i: (0, 0)),
          ]),
      out_specs=[pl.BlockSpec((BN, out_w), lambda i: (i, 0))
                 for _ in range(nout)],
      out_shape=[jax.ShapeDtypeStruct((N, out_w), jnp.float32)
                 for _ in range(nout)],
  )(*S_list, *hs_list, dinv, W, b)


def _gate_tc(h3, gate_W, gate_b, bidx):
  def body(h_ref, gw_ref, gb_ref, bi_ref, gate_ref, m_ref):
    i = pl.program_id(0)
    g = jnp.dot(h_ref[...], gw_ref[...],
                preferred_element_type=jnp.float32) + gb_ref[...]
    gate_ref[...] = g
    gids = lax.broadcasted_iota(jnp.int32, (1, G), 1)
    mask = bi_ref[...] == gids
    cm = jnp.max(jnp.where(mask, g, -1e30), axis=0)[:, None]

    @pl.when(i == 0)
    def _():
      m_ref[...] = jnp.full((G, 1), -1e30, jnp.float32)

    m_ref[...] = jnp.maximum(m_ref[...], cm)

  return pl.pallas_call(
      body,
      grid=(N // BN,),
      in_specs=[
          pl.BlockSpec((BN, F2), lambda i: (i, 0)),
          pl.BlockSpec((F2, 1), lambda i: (0, 0)),
          pl.BlockSpec((1, 1), lambda i: (0, 0)),
          pl.BlockSpec((BN, 1), lambda i: (i, 0)),
      ],
      out_specs=[
          pl.BlockSpec((BN, 1), lambda i: (i, 0)),
          pl.BlockSpec((G, 1), lambda i: (0, 0)),
      ],
      out_shape=[
          jax.ShapeDtypeStruct((N, 1), jnp.float32),
          jax.ShapeDtypeStruct((G, 1), jnp.float32),
      ],
  )(h3, gate_W, gate_b, bidx)


def _pool_tc(h3, gate, m, bidx):
  def body(h_ref, gate_ref, m_ref, bi_ref, sums_ref, den_ref):
    i = pl.program_id(0)
    gids = lax.broadcasted_iota(jnp.int32, (1, G), 1)
    maskf = (bi_ref[...] == gids).astype(jnp.float32)
    m_sel = jnp.dot(maskf, m_ref[...], preferred_element_type=jnp.float32)
    e = jnp.exp(gate_ref[...] - m_sel)
    A = maskf * e
    dc = jnp.sum(A, axis=0)[:, None]
    sc = lax.dot_general(A, h_ref[...], (((0,), (0,)), ((), ())),
                         preferred_element_type=jnp.float32)

    @pl.when(i == 0)
    def _():
      sums_ref[...] = jnp.zeros_like(sums_ref)
      den_ref[...] = jnp.zeros_like(den_ref)

    sums_ref[...] += sc
    den_ref[...] += dc

  return pl.pallas_call(
      body,
      grid=(N // BN,),
      in_specs=[
          pl.BlockSpec((BN, F2), lambda i: (i, 0)),
          pl.BlockSpec((BN, 1), lambda i: (i, 0)),
          pl.BlockSpec((G, 1), lambda i: (0, 0)),
          pl.BlockSpec((BN, 1), lambda i: (i, 0)),
      ],
      out_specs=[
          pl.BlockSpec((G, F2), lambda i: (0, 0)),
          pl.BlockSpec((G, 1), lambda i: (0, 0)),
      ],
      out_shape=[
          jax.ShapeDtypeStruct((G, F2), jnp.float32),
          jax.ShapeDtypeStruct((G, 1), jnp.float32),
      ],
  )(h3, gate, m, bidx)


def _head_tc(sums, den, W_ih0, b_ih0, b_hh0, W_ih1, b_ih1, b_hh1,
             proj_W, proj_b):
  def cell(xin, W_ih, b_ih, b_hh):
    gi = lax.dot_general(xin, W_ih, (((1,), (1,)), ((), ())),
                         preferred_element_type=jnp.float32) + b_ih
    i_r, i_z, i_n = gi[:, :GH], gi[:, GH:2 * GH], gi[:, 2 * GH:]
    h_r, h_z, h_n = b_hh[:, :GH], b_hh[:, GH:2 * GH], b_hh[:, 2 * GH:]
    r = jax.nn.sigmoid(i_r + h_r)
    z = jax.nn.sigmoid(i_z + h_z)
    n = jnp.tanh(i_n + r * h_n)
    return (1.0 - z) * n

  def body(sums_ref, den_ref, wi0_ref, bi0_ref, bh0_ref,
           wi1_ref, bi1_ref, bh1_ref, pw_ref, pb_ref, out_ref):
    emb = sums_ref[...] / (den_ref[...] + 1e-16)
    g0 = cell(emb, wi0_ref[...], bi0_ref[...], bh0_ref[...])
    g1 = cell(g0, wi1_ref[...], bi1_ref[...], bh1_ref[...])
    out_ref[...] = jnp.dot(g1, pw_ref[...],
                           preferred_element_type=jnp.float32) + pb_ref[...]

  return pl.pallas_call(
      body,
      out_shape=jax.ShapeDtypeStruct((G, F2), jnp.float32),
  )(sums, den, W_ih0, b_ih0, b_hh0, W_ih1, b_ih1, b_hh1, proj_W, proj_b)


def kernel(x, edge_index, batch_idx, W0, b0, W1, b1, W2, b2, gate_W, gate_b,
           W_ih0, W_hh0, b_ih0, b_hh0, W_ih1, W_hh1, b_ih1, b_hh1,
           proj_W, proj_b):
  pad = jnp.stack([jnp.zeros((EPAD,), jnp.int32),
                   jnp.full((EPAD,), NP - 1, jnp.int32)])
  ep = jnp.concatenate([edge_index, pad], axis=1)
  eblk = ep.reshape(2, NBLKP, EB).transpose(1, 0, 2)

  deg2 = _deg_call(ep)
  us0, dinv = _pre0_tc(jnp.transpose(deg2), x, W0)

  S0 = _agg128(us0, eblk)
  hs1 = _mid0_tc(S0, us0, dinv, b0[None, :])
  S1 = _agg128(hs1, eblk)
  hs2a, hs2b = _gcn_tc([S1], [hs1], dinv, W1, b1[None, :], scale_out=True)
  S2a = _agg128(hs2a, eblk)
  S2b = _agg128(hs2b, eblk)
  (h3,) = _gcn_tc([S2a, S2b], [hs2a, hs2b], dinv, W2, b2[None, :],
                  scale_out=False)

  bidx = batch_idx[:, None]
  gate, m = _gate_tc(h3, gate_W, gate_b[None, :], bidx)
  sums, den = _pool_tc(h3, gate, m, bidx)
  return _head_tc(sums, den, W_ih0, b_ih0[None, :], b_hh0[None, :],
                  W_ih1, b_ih1[None, :], b_hh1[None, :], proj_W,
                  proj_b[None, :])


# speedup vs baseline: 1.0202x; 1.0202x over previous
import functools

import jax
import jax.numpy as jnp
from jax import lax
from jax.experimental import pallas as pl
from jax.experimental.pallas import tpu as pltpu
from jax.experimental.pallas import tpu_sc as plsc

N = 50000
E = 800000
G = 256
DIN = 64
F0, F1, F2 = 128, 256, 512
GH = 256

NC, NS = 2, 16
EB = 128
NBLK = E // EB
CW = 128
NP = 50688
HSEG = NP // NS
EPAD = 2816
NBLKP = (E + EPAD) // EB

BN = 2000


def _make_deg_kernel():
  bp = NBLKP // (NC * NS)

  @functools.partial(
      pl.kernel,
      out_type=jax.ShapeDtypeStruct((NC, NP), jnp.float32),
      mesh=plsc.VectorSubcoreMesh(core_axis_name="c", subcore_axis_name="s"),
      scratch_types=[
          pltpu.VMEM((NP,), jnp.float32),
          pltpu.VMEM((EB,), jnp.int32),
          pltpu.VMEM((HSEG,), jnp.float32),
          pltpu.VMEM((HSEG,), jnp.float32),
          pltpu.VMEM_SHARED((NS, NP), jnp.float32),
      ],
      compiler_params=pltpu.CompilerParams(
          use_tc_tiling_on_sc=False, needs_layout_passes=False),
  )
  def deg_kernel(edges_hbm, out_hbm, hist, didx, acc, tmp, shared):
    c = lax.axis_index("c")
    s = lax.axis_index("s")
    wid = c * NS + s
    zeros16 = jnp.zeros((16,), jnp.float32)
    ones16 = jnp.ones((16,), jnp.float32)

    def zero_hist(i, carry):
      hist[pl.ds(i * 16, 16)] = zeros16
      return carry
    lax.fori_loop(0, NP // 16, zero_hist, 0)

    base = wid * bp

    def blk_body(b, carry):
      off = (base + b) * EB
      pltpu.sync_copy(edges_hbm.at[1, pl.ds(off, EB)], didx)
      def lane_body(j, carry2):
        idx = didx[pl.ds(j * 16, 16)]
        plsc.addupdate_scatter(hist, [idx], ones16)
        return carry2
      lax.fori_loop(0, EB // 16, lane_body, 0)
      return carry
    lax.fori_loop(0, bp, blk_body, 0)

    pltpu.sync_copy(hist, shared.at[s])
    plsc.subcore_barrier()

    def zero_acc(i, carry):
      acc[pl.ds(i * 16, 16)] = zeros16
      return carry
    lax.fori_loop(0, HSEG // 16, zero_acc, 0)
    seg0 = s * HSEG
    for t in range(NS):
      pltpu.sync_copy(shared.at[t, pl.ds(seg0, HSEG)], tmp)
      def radd(i, carry):
        acc[pl.ds(i * 16, 16)] = acc[pl.ds(i * 16, 16)] + tmp[pl.ds(i * 16, 16)]
        return carry
      lax.fori_loop(0, HSEG // 16, radd, 0)
    pltpu.sync_copy(acc, out_hbm.at[c, pl.ds(seg0, HSEG)])

  return deg_kernel


def _make_agg_kernel():
  fin = 128
  NSL = 2
  RNP = 8448
  ZB = 176
  SPS = RNP // NS
  npass = NP // RNP // NC
  bp = NBLKP // NS

  @functools.partial(
      pl.kernel,
      out_type=jax.ShapeDtypeStruct((NP, fin), jnp.float32),
      mesh=plsc.VectorSubcoreMesh(core_axis_name="c", subcore_axis_name="s"),
      scratch_types=[
          pltpu.VMEM((ZB, fin), jnp.float32),
          [pltpu.VMEM((EB,), jnp.int32)] * NSL,
          [pltpu.VMEM((EB,), jnp.int32)] * NSL,
          [pltpu.VMEM((EB,), jnp.int32)] * NSL,
          [pltpu.VMEM((EB,), jnp.int32)] * NSL,
          [pltpu.VMEM((EB, fin), jnp.float32)] * NSL,
          pltpu.VMEM_SHARED((RNP, fin), jnp.float32),
          [pltpu.SemaphoreType.DMA] * NSL,
          [pltpu.SemaphoreType.DMA] * NSL,
          [pltpu.SemaphoreType.DMA] * NSL,
      ],
  )
  def agg_kernel(hs_hbm, edges_hbm, out_hbm, zbuf, sv2, dv2, gi2, si2, rw2,
                 accum, isem, gsem, ssem):
    c = lax.axis_index("c")
    s = lax.axis_index("s")
    zeros16 = jnp.zeros((16,), jnp.float32)

    def zb_body(i, carry):
      for q in range(fin // 16):
        zbuf[i, pl.ds(q * 16, 16)] = zeros16
      return carry
    lax.fori_loop(0, ZB, zb_body, 0)

    bbase = s * bp

    def issue_idx(b, sl):
      off = (bbase + jnp.minimum(b, bp - 1)) * EB
      pltpu.async_copy(edges_hbm.at[0, pl.ds(off, EB)], sv2[sl], isem[sl])
      pltpu.async_copy(edges_hbm.at[1, pl.ds(off, EB)], dv2[sl], isem[sl])

    def wait_idx(sl):
      pltpu.make_async_copy(edges_hbm.at[0, pl.ds(0, EB)], sv2[sl],
                            isem[sl]).wait()
      pltpu.make_async_copy(edges_hbm.at[1, pl.ds(0, EB)], dv2[sl],
                            isem[sl]).wait()

    for pi in range(npass):
      nbase = (c * npass + pi) * RNP

      for z in range(SPS // ZB):
        pltpu.sync_copy(zbuf, accum.at[pl.ds(s * SPS + z * ZB, ZB)])
      plsc.subcore_barrier()

      def compute_masks(sl, nb):
        def lane_body(j, carry2):
          sv = sv2[sl][pl.ds(j * 16, 16)]
          dv = dv2[sl][pl.ds(j * 16, 16)]
          inr = (dv >= nb) & (dv < nb + RNP)
          gi2[sl][pl.ds(j * 16, 16)] = jnp.where(inr, sv, -1)
          si2[sl][pl.ds(j * 16, 16)] = jnp.where(inr, dv - nb, -1)
          return carry2
        lax.fori_loop(0, EB // 16, lane_body, 0)

      def start_gather(sl):
        pltpu.async_copy(hs_hbm.at[plsc.Indices(gi2[sl], ignored_value=-1)],
                         rw2[sl], gsem[sl])

      def wait_gather(sl):
        pltpu.make_async_copy(
            hs_hbm.at[plsc.Indices(gi2[sl], ignored_value=-1)], rw2[sl],
            gsem[sl]).wait()

      def start_scatter(sl):
        pltpu.async_copy(rw2[sl],
                         accum.at[plsc.Indices(si2[sl], ignored_value=-1)],
                         ssem[sl], add=True)

      def wait_scatter(sl):
        pltpu.make_async_copy(rw2[sl],
                              accum.at[plsc.Indices(si2[sl],
                                                    ignored_value=-1)],
                              ssem[sl]).wait()

      for sl in range(NSL):
        issue_idx(sl, sl)

      def grp_body(g, carry):
        b0 = NSL * g
        for sl in range(NSL):
          wait_idx(sl)

          @pl.when(g > 0)
          def _():
            wait_scatter(sl)
          compute_masks(sl, nbase)
          start_gather(sl)
          issue_idx(b0 + NSL + sl, sl)
        for sl in range(NSL):
          wait_gather(sl)
          start_scatter(sl)
        return carry
      lax.fori_loop(0, bp // NSL, grp_body, 0)

      for sl in range(NSL):
        wait_scatter(sl)
        wait_idx(sl)
      plsc.subcore_barrier()

      for z in range(SPS // ZB):
        r0 = s * SPS + z * ZB
        pltpu.sync_copy(accum.at[pl.ds(r0, ZB)],
                        out_hbm.at[pl.ds(nbase + r0, ZB)])
      plsc.subcore_barrier()

  return agg_kernel


_deg_call = _make_deg_kernel()
_agg128 = _make_agg_kernel()


def _pre0_tc(deg2t, x, W0):
  def body(deg_ref, x_ref, w_ref, us_ref, dinv_ref):
    d = deg_ref[:, 0] + deg_ref[:, 1] + 1.0
    dv = lax.rsqrt(d)[:, None]
    dinv_ref[...] = dv
    u = jnp.dot(x_ref[...], w_ref[...], preferred_element_type=jnp.float32)
    us_ref[...] = u * dv

  return pl.pallas_call(
      body,
      grid=(N // BN,),
      in_specs=[
          pl.BlockSpec((BN, 2), lambda i: (i, 0)),
          pl.BlockSpec((BN, DIN), lambda i: (i, 0)),
          pl.BlockSpec((DIN, F0), lambda i: (0, 0)),
      ],
      out_specs=[
          pl.BlockSpec((BN, F0), lambda i: (i, 0)),
          pl.BlockSpec((BN, 1), lambda i: (i, 0)),
      ],
      out_shape=[
          jax.ShapeDtypeStruct((N, F0), jnp.float32),
          jax.ShapeDtypeStruct((N, 1), jnp.float32),
      ],
  )(deg2t, x, W0)


def _mid0_tc(S0, us0, dinv, b0):
  def body(S_ref, us_ref, dinv_ref, b_ref, out_ref):
    dv = dinv_ref[...]
    h = jnp.maximum((S_ref[...] + us_ref[...]) * dv + b_ref[...], 0.0)
    out_ref[...] = h * dv

  return pl.pallas_call(
      body,
      grid=(N // BN,),
      in_specs=[
          pl.BlockSpec((BN, F0), lambda i: (i, 0)),
          pl.BlockSpec((BN, F0), lambda i: (i, 0)),
          pl.BlockSpec((BN, 1), lambda i: (i, 0)),
          pl.BlockSpec((1, F0), lambda i: (0, 0)),
      ],
      out_specs=pl.BlockSpec((BN, F0), lambda i: (i, 0)),
      out_shape=jax.ShapeDtypeStruct((N, F0), jnp.float32),
  )(S0, us0, dinv, b0)


def _gcn_tc(S_list, hs_list, dinv, W, b, scale_out):
  fin, fout = W.shape
  nin = len(S_list)
  assert nin * 128 == fin and len(hs_list) == nin
  nout = fout // 128 if scale_out else 1

  def body(*refs):
    S_refs = refs[:nin]
    hs_refs = refs[nin:2 * nin]
    dinv_ref, W_ref, b_ref = refs[2 * nin:2 * nin + 3]
    out_refs = refs[2 * nin + 3:]
    dv = dinv_ref[...]
    if nin == 1:
      t = (S_refs[0][...] + hs_refs[0][...]) * dv
    else:
      t = jnp.concatenate(
          [S_refs[q][...] + hs_refs[q][...] for q in range(nin)], axis=1) * dv
    t = jnp.dot(t, W_ref[...], preferred_element_type=jnp.float32) + b_ref[...]
    h = jnp.maximum(t, 0.0)
    if scale_out:
      h = h * dv
      for q in range(nout):
        out_refs[q][...] = h[:, q * 128:(q + 1) * 128]
    else:
      out_refs[0][...] = h

  piece = lambda: pl.BlockSpec((BN, 128), lambda i: (i, 0))
  out_w = 128 if scale_out else fout
  return pl.pallas_call(
      body,
      grid=(N // BN,),
      in_specs=(
          [piece() for _ in range(2 * nin)] + [
              pl.BlockSpec((BN, 1), lambda i: (i, 0)),
              pl.BlockSpec((fin, fout), lambda i: (0, 0)),
              pl.BlockSpec((1, fout), lambda i: (0, 0)),
          ]),
      out_specs=[pl.BlockSpec((BN, out_w), lambda i: (i, 0))
                 for _ in range(nout)],
      out_shape=[jax.ShapeDtypeStruct((N, out_w), jnp.float32)
                 for _ in range(nout)],
  )(*S_list, *hs_list, dinv, W, b)


def _gate_tc(h3, gate_W, gate_b, bidx):
  def body(h_ref, gw_ref, gb_ref, bi_ref, gate_ref, m_ref):
    i = pl.program_id(0)
    g = jnp.dot(h_ref[...], gw_ref[...],
                preferred_element_type=jnp.float32) + gb_ref[...]
    gate_ref[...] = g
    gids = lax.broadcasted_iota(jnp.int32, (1, G), 1)
    mask = bi_ref[...] == gids
    cm = jnp.max(jnp.where(mask, g, -1e30), axis=0)[:, None]

    @pl.when(i == 0)
    def _():
      m_ref[...] = jnp.full((G, 1), -1e30, jnp.float32)

    m_ref[...] = jnp.maximum(m_ref[...], cm)

  return pl.pallas_call(
      body,
      grid=(N // BN,),
      in_specs=[
          pl.BlockSpec((BN, F2), lambda i: (i, 0)),
          pl.BlockSpec((F2, 1), lambda i: (0, 0)),
          pl.BlockSpec((1, 1), lambda i: (0, 0)),
          pl.BlockSpec((BN, 1), lambda i: (i, 0)),
      ],
      out_specs=[
          pl.BlockSpec((BN, 1), lambda i: (i, 0)),
          pl.BlockSpec((G, 1), lambda i: (0, 0)),
      ],
      out_shape=[
          jax.ShapeDtypeStruct((N, 1), jnp.float32),
          jax.ShapeDtypeStruct((G, 1), jnp.float32),
      ],
  )(h3, gate_W, gate_b, bidx)


def _pool_tc(h3, gate, m, bidx):
  def body(h_ref, gate_ref, m_ref, bi_ref, sums_ref, den_ref):
    i = pl.program_id(0)
    gids = lax.broadcasted_iota(jnp.int32, (1, G), 1)
    maskf = (bi_ref[...] == gids).astype(jnp.float32)
    m_sel = jnp.dot(maskf, m_ref[...], preferred_element_type=jnp.float32)
    e = jnp.exp(gate_ref[...] - m_sel)
    A = maskf * e
    dc = jnp.sum(A, axis=0)[:, None]
    sc = lax.dot_general(A, h_ref[...], (((0,), (0,)), ((), ())),
                         preferred_element_type=jnp.float32)

    @pl.when(i == 0)
    def _():
      sums_ref[...] = jnp.zeros_like(sums_ref)
      den_ref[...] = jnp.zeros_like(den_ref)

    sums_ref[...] += sc
    den_ref[...] += dc

  return pl.pallas_call(
      body,
      grid=(N // BN,),
      in_specs=[
          pl.BlockSpec((BN, F2), lambda i: (i, 0)),
          pl.BlockSpec((BN, 1), lambda i: (i, 0)),
          pl.BlockSpec((G, 1), lambda i: (0, 0)),
          pl.BlockSpec((BN, 1), lambda i: (i, 0)),
      ],
      out_specs=[
          pl.BlockSpec((G, F2), lambda i: (0, 0)),
          pl.BlockSpec((G, 1), lambda i: (0, 0)),
      ],
      out_shape=[
          jax.ShapeDtypeStruct((G, F2), jnp.float32),
          jax.ShapeDtypeStruct((G, 1), jnp.float32),
      ],
  )(h3, gate, m, bidx)


def _head_tc(sums, den, W_ih0, b_ih0, b_hh0, W_ih1, b_ih1, b_hh1,
             proj_W, proj_b):
  def cell(xin, W_ih, b_ih, b_hh):
    gi = lax.dot_general(xin, W_ih, (((1,), (1,)), ((), ())),
                         preferred_element_type=jnp.float32) + b_ih
    i_r, i_z, i_n = gi[:, :GH], gi[:, GH:2 * GH], gi[:, 2 * GH:]
    h_r, h_z, h_n = b_hh[:, :GH], b_hh[:, GH:2 * GH], b_hh[:, 2 * GH:]
    r = jax.nn.sigmoid(i_r + h_r)
    z = jax.nn.sigmoid(i_z + h_z)
    n = jnp.tanh(i_n + r * h_n)
    return (1.0 - z) * n

  def body(sums_ref, den_ref, wi0_ref, bi0_ref, bh0_ref,
           wi1_ref, bi1_ref, bh1_ref, pw_ref, pb_ref, out_ref):
    emb = sums_ref[...] / (den_ref[...] + 1e-16)
    g0 = cell(emb, wi0_ref[...], bi0_ref[...], bh0_ref[...])
    g1 = cell(g0, wi1_ref[...], bi1_ref[...], bh1_ref[...])
    out_ref[...] = jnp.dot(g1, pw_ref[...],
                           preferred_element_type=jnp.float32) + pb_ref[...]

  return pl.pallas_call(
      body,
      out_shape=jax.ShapeDtypeStruct((G, F2), jnp.float32),
  )(sums, den, W_ih0, b_ih0, b_hh0, W_ih1, b_ih1, b_hh1, proj_W, proj_b)


def kernel(x, edge_index, batch_idx, W0, b0, W1, b1, W2, b2, gate_W, gate_b,
           W_ih0, W_hh0, b_ih0, b_hh0, W_ih1, W_hh1, b_ih1, b_hh1,
           proj_W, proj_b):
  pad = jnp.stack([jnp.zeros((EPAD,), jnp.int32),
                   jnp.full((EPAD,), NP - 1, jnp.int32)])
  ep = jnp.concatenate([edge_index, pad], axis=1)

  deg2 = _deg_call(ep)
  us0, dinv = _pre0_tc(jnp.transpose(deg2), x, W0)

  S0 = _agg128(us0, ep)
  hs1 = _mid0_tc(S0, us0, dinv, b0[None, :])
  S1 = _agg128(hs1, ep)
  hs2a, hs2b = _gcn_tc([S1], [hs1], dinv, W1, b1[None, :], scale_out=True)
  S2a = _agg128(hs2a, ep)
  S2b = _agg128(hs2b, ep)
  (h3,) = _gcn_tc([S2a, S2b], [hs2a, hs2b], dinv, W2, b2[None, :],
                  scale_out=False)

  bidx = batch_idx[:, None]
  gate, m = _gate_tc(h3, gate_W, gate_b[None, :], bidx)
  sums, den = _pool_tc(h3, gate, m, bidx)
  return _head_tc(sums, den, W_ih0, b_ih0[None, :], b_hh0[None, :],
                  W_ih1, b_ih1[None, :], b_hh1[None, :], proj_W,
                  proj_b[None, :])


# speedup vs baseline: 1.5462x; 1.5156x over previous
import functools

import jax
import jax.numpy as jnp
from jax import lax
from jax.experimental import pallas as pl
from jax.experimental.pallas import tpu as pltpu
from jax.experimental.pallas import tpu_sc as plsc

N = 50000
E = 800000
G = 256
DIN = 64
F0, F1, F2 = 128, 256, 512
GH = 256

NC, NS = 2, 16
EB = 128
NBLK = E // EB
CW = 128
NP = 50688
HSEG = NP // NS
EPAD = 2816
NBLKP = (E + EPAD) // EB

BN = 2000


def _make_deg_kernel():
  bp = NBLKP // (NC * NS)

  @functools.partial(
      pl.kernel,
      out_type=jax.ShapeDtypeStruct((NC, NP), jnp.float32),
      mesh=plsc.VectorSubcoreMesh(core_axis_name="c", subcore_axis_name="s"),
      scratch_types=[
          pltpu.VMEM((NP,), jnp.float32),
          pltpu.VMEM((EB,), jnp.int32),
          pltpu.VMEM((HSEG,), jnp.float32),
          pltpu.VMEM((HSEG,), jnp.float32),
          pltpu.VMEM_SHARED((NS, NP), jnp.float32),
      ],
      compiler_params=pltpu.CompilerParams(
          use_tc_tiling_on_sc=False, needs_layout_passes=False),
  )
  def deg_kernel(edges_hbm, out_hbm, hist, didx, acc, tmp, shared):
    c = lax.axis_index("c")
    s = lax.axis_index("s")
    wid = c * NS + s
    zeros16 = jnp.zeros((16,), jnp.float32)
    ones16 = jnp.ones((16,), jnp.float32)

    def zero_hist(i, carry):
      hist[pl.ds(i * 16, 16)] = zeros16
      return carry
    lax.fori_loop(0, NP // 16, zero_hist, 0)

    base = wid * bp

    def blk_body(b, carry):
      off = (base + b) * EB
      pltpu.sync_copy(edges_hbm.at[1, pl.ds(off, EB)], didx)
      def lane_body(j, carry2):
        idx = didx[pl.ds(j * 16, 16)]
        plsc.addupdate_scatter(hist, [idx], ones16)
        return carry2
      lax.fori_loop(0, EB // 16, lane_body, 0)
      return carry
    lax.fori_loop(0, bp, blk_body, 0)

    pltpu.sync_copy(hist, shared.at[s])
    plsc.subcore_barrier()

    def zero_acc(i, carry):
      acc[pl.ds(i * 16, 16)] = zeros16
      return carry
    lax.fori_loop(0, HSEG // 16, zero_acc, 0)
    seg0 = s * HSEG
    for t in range(NS):
      pltpu.sync_copy(shared.at[t, pl.ds(seg0, HSEG)], tmp)
      def radd(i, carry):
        acc[pl.ds(i * 16, 16)] = acc[pl.ds(i * 16, 16)] + tmp[pl.ds(i * 16, 16)]
        return carry
      lax.fori_loop(0, HSEG // 16, radd, 0)
    pltpu.sync_copy(acc, out_hbm.at[c, pl.ds(seg0, HSEG)])

  return deg_kernel


def _make_agg_kernel():
  fin = 128
  NSL = 2
  RNP = 8448
  ZB = 176
  SPS = RNP // NS
  npass = NP // RNP // NC
  bp = NBLKP // NS

  @functools.partial(
      pl.kernel,
      out_type=jax.ShapeDtypeStruct((NP, fin), jnp.float32),
      mesh=plsc.VectorSubcoreMesh(core_axis_name="c", subcore_axis_name="s"),
      scratch_types=[
          pltpu.VMEM((ZB, fin), jnp.float32),
          [pltpu.VMEM((EB,), jnp.int32)] * NSL,
          [pltpu.VMEM((EB,), jnp.int32)] * NSL,
          [pltpu.VMEM((EB,), jnp.int32)] * NSL,
          [pltpu.VMEM((EB,), jnp.int32)] * NSL,
          [pltpu.VMEM((EB, fin), jnp.float32)] * NSL,
          pltpu.VMEM_SHARED((RNP, fin), jnp.float32),
          [pltpu.SemaphoreType.DMA] * NSL,
          [pltpu.SemaphoreType.DMA] * NSL,
          [pltpu.SemaphoreType.DMA] * NSL,
      ],
  )
  def agg_kernel(hs_hbm, edges_hbm, out_hbm, zbuf, sv2, dv2, gi2, si2, rw2,
                 accum, isem, gsem, ssem):
    c = lax.axis_index("c")
    s = lax.axis_index("s")
    zeros16 = jnp.zeros((16,), jnp.float32)

    def zb_body(i, carry):
      for q in range(fin // 16):
        zbuf[i, pl.ds(q * 16, 16)] = zeros16
      return carry
    lax.fori_loop(0, ZB, zb_body, 0)

    bbase = s * bp

    def issue_idx(b, sl):
      off = (bbase + jnp.minimum(b, bp - 1)) * EB
      pltpu.async_copy(edges_hbm.at[0, pl.ds(off, EB)], sv2[sl], isem[sl])
      pltpu.async_copy(edges_hbm.at[1, pl.ds(off, EB)], dv2[sl], isem[sl])

    def wait_idx(sl):
      pltpu.make_async_copy(edges_hbm.at[0, pl.ds(0, EB)], sv2[sl],
                            isem[sl]).wait()
      pltpu.make_async_copy(edges_hbm.at[1, pl.ds(0, EB)], dv2[sl],
                            isem[sl]).wait()

    for pi in range(npass):
      nbase = (c * npass + pi) * RNP

      for z in range(SPS // ZB):
        pltpu.sync_copy(zbuf, accum.at[pl.ds(s * SPS + z * ZB, ZB)])
      plsc.subcore_barrier()

      def compute_masks(sl, nb):
        for j in range(EB // 16):
          sv = sv2[sl][pl.ds(j * 16, 16)]
          dv = dv2[sl][pl.ds(j * 16, 16)]
          du = dv - nb
          inr = plsc.bitcast(du, jnp.uint32) < jnp.uint32(RNP)
          gi2[sl][pl.ds(j * 16, 16)] = jnp.where(inr, sv, -1)
          si2[sl][pl.ds(j * 16, 16)] = jnp.where(inr, du, -1)

      def start_gather(sl):
        pltpu.async_copy(hs_hbm.at[plsc.Indices(gi2[sl], ignored_value=-1)],
                         rw2[sl], gsem[sl])

      def wait_gather(sl):
        pltpu.make_async_copy(
            hs_hbm.at[plsc.Indices(gi2[sl], ignored_value=-1)], rw2[sl],
            gsem[sl]).wait()

      def start_scatter(sl):
        pltpu.async_copy(rw2[sl],
                         accum.at[plsc.Indices(si2[sl], ignored_value=-1)],
                         ssem[sl], add=True)

      def wait_scatter(sl):
        pltpu.make_async_copy(rw2[sl],
                              accum.at[plsc.Indices(si2[sl],
                                                    ignored_value=-1)],
                              ssem[sl]).wait()

      for sl in range(NSL):
        issue_idx(sl, sl)

      def grp_body(g, carry):
        b0 = NSL * g
        for sl in range(NSL):
          wait_idx(sl)

          @pl.when(g > 0)
          def _():
            wait_scatter(sl)
          compute_masks(sl, nbase)
          start_gather(sl)
          issue_idx(b0 + NSL + sl, sl)
        for sl in range(NSL):
          wait_gather(sl)
          start_scatter(sl)
        return carry
      lax.fori_loop(0, bp // NSL, grp_body, 0)

      for sl in range(NSL):
        wait_scatter(sl)
        wait_idx(sl)
      plsc.subcore_barrier()

      for z in range(SPS // ZB):
        r0 = s * SPS + z * ZB
        pltpu.sync_copy(accum.at[pl.ds(r0, ZB)],
                        out_hbm.at[pl.ds(nbase + r0, ZB)])
      plsc.subcore_barrier()

  return agg_kernel


_deg_call = _make_deg_kernel()
_agg128 = _make_agg_kernel()


def _pre0_tc(deg2t, x, W0):
  def body(deg_ref, x_ref, w_ref, us_ref, dinv_ref):
    d = deg_ref[:, 0] + deg_ref[:, 1] + 1.0
    dv = lax.rsqrt(d)[:, None]
    dinv_ref[...] = dv
    u = jnp.dot(x_ref[...], w_ref[...], preferred_element_type=jnp.float32)
    us_ref[...] = u * dv

  return pl.pallas_call(
      body,
      grid=(N // BN,),
      in_specs=[
          pl.BlockSpec((BN, 2), lambda i: (i, 0)),
          pl.BlockSpec((BN, DIN), lambda i: (i, 0)),
          pl.BlockSpec((DIN, F0), lambda i: (0, 0)),
      ],
      out_specs=[
          pl.BlockSpec((BN, F0), lambda i: (i, 0)),
          pl.BlockSpec((BN, 1), lambda i: (i, 0)),
      ],
      out_shape=[
          jax.ShapeDtypeStruct((N, F0), jnp.float32),
          jax.ShapeDtypeStruct((N, 1), jnp.float32),
      ],
  )(deg2t, x, W0)


def _mid0_tc(S0, us0, dinv, b0):
  def body(S_ref, us_ref, dinv_ref, b_ref, out_ref):
    dv = dinv_ref[...]
    h = jnp.maximum((S_ref[...] + us_ref[...]) * dv + b_ref[...], 0.0)
    out_ref[...] = h * dv

  return pl.pallas_call(
      body,
      grid=(N // BN,),
      in_specs=[
          pl.BlockSpec((BN, F0), lambda i: (i, 0)),
          pl.BlockSpec((BN, F0), lambda i: (i, 0)),
          pl.BlockSpec((BN, 1), lambda i: (i, 0)),
          pl.BlockSpec((1, F0), lambda i: (0, 0)),
      ],
      out_specs=pl.BlockSpec((BN, F0), lambda i: (i, 0)),
      out_shape=jax.ShapeDtypeStruct((N, F0), jnp.float32),
  )(S0, us0, dinv, b0)


def _gcn_tc(S_list, hs_list, dinv, W, b, scale_out):
  fin, fout = W.shape
  nin = len(S_list)
  assert nin * 128 == fin and len(hs_list) == nin
  nout = fout // 128 if scale_out else 1

  def body(*refs):
    S_refs = refs[:nin]
    hs_refs = refs[nin:2 * nin]
    dinv_ref, W_ref, b_ref = refs[2 * nin:2 * nin + 3]
    out_refs = refs[2 * nin + 3:]
    dv = dinv_ref[...]
    if nin == 1:
      t = (S_refs[0][...] + hs_refs[0][...]) * dv
    else:
      t = jnp.concatenate(
          [S_refs[q][...] + hs_refs[q][...] for q in range(nin)], axis=1) * dv
    t = jnp.dot(t, W_ref[...], preferred_element_type=jnp.float32) + b_ref[...]
    h = jnp.maximum(t, 0.0)
    if scale_out:
      h = h * dv
      for q in range(nout):
        out_refs[q][...] = h[:, q * 128:(q + 1) * 128]
    else:
      out_refs[0][...] = h

  piece = lambda: pl.BlockSpec((BN, 128), lambda i: (i, 0))
  out_w = 128 if scale_out else fout
  return pl.pallas_call(
      body,
      grid=(N // BN,),
      in_specs=(
          [piece() for _ in range(2 * nin)] + [
              pl.BlockSpec((BN, 1), lambda i: (i, 0)),
              pl.BlockSpec((fin, fout), lambda i: (0, 0)),
              pl.BlockSpec((1, fout), lambda i: (0, 0)),
          ]),
      out_specs=[pl.BlockSpec((BN, out_w), lambda i: (i, 0))
                 for _ in range(nout)],
      out_shape=[jax.ShapeDtypeStruct((N, out_w), jnp.float32)
                 for _ in range(nout)],
  )(*S_list, *hs_list, dinv, W, b)


def _gate_tc(h3, gate_W, gate_b, bidx):
  def body(h_ref, gw_ref, gb_ref, bi_ref, gate_ref, m_ref):
    i = pl.program_id(0)
    g = jnp.dot(h_ref[...], gw_ref[...],
                preferred_element_type=jnp.float32) + gb_ref[...]
    gate_ref[...] = g
    gids = lax.broadcasted_iota(jnp.int32, (1, G), 1)
    mask = bi_ref[...] == gids
    cm = jnp.max(jnp.where(mask, g, -1e30), axis=0)[:, None]

    @pl.when(i == 0)
    def _():
      m_ref[...] = jnp.full((G, 1), -1e30, jnp.float32)

    m_ref[...] = jnp.maximum(m_ref[...], cm)

  return pl.pallas_call(
      body,
      grid=(N // BN,),
      in_specs=[
          pl.BlockSpec((BN, F2), lambda i: (i, 0)),
          pl.BlockSpec((F2, 1), lambda i: (0, 0)),
          pl.BlockSpec((1, 1), lambda i: (0, 0)),
          pl.BlockSpec((BN, 1), lambda i: (i, 0)),
      ],
      out_specs=[
          pl.BlockSpec((BN, 1), lambda i: (i, 0)),
          pl.BlockSpec((G, 1), lambda i: (0, 0)),
      ],
      out_shape=[
          jax.ShapeDtypeStruct((N, 1), jnp.float32),
          jax.ShapeDtypeStruct((G, 1), jnp.float32),
      ],
  )(h3, gate_W, gate_b, bidx)


def _pool_tc(h3, gate, m, bidx):
  def body(h_ref, gate_ref, m_ref, bi_ref, sums_ref, den_ref):
    i = pl.program_id(0)
    gids = lax.broadcasted_iota(jnp.int32, (1, G), 1)
    maskf = (bi_ref[...] == gids).astype(jnp.float32)
    m_sel = jnp.dot(maskf, m_ref[...], preferred_element_type=jnp.float32)
    e = jnp.exp(gate_ref[...] - m_sel)
    A = maskf * e
    dc = jnp.sum(A, axis=0)[:, None]
    sc = lax.dot_general(A, h_ref[...], (((0,), (0,)), ((), ())),
                         preferred_element_type=jnp.float32)

    @pl.when(i == 0)
    def _():
      sums_ref[...] = jnp.zeros_like(sums_ref)
      den_ref[...] = jnp.zeros_like(den_ref)

    sums_ref[...] += sc
    den_ref[...] += dc

  return pl.pallas_call(
      body,
      grid=(N // BN,),
      in_specs=[
          pl.BlockSpec((BN, F2), lambda i: (i, 0)),
          pl.BlockSpec((BN, 1), lambda i: (i, 0)),
          pl.BlockSpec((G, 1), lambda i: (0, 0)),
          pl.BlockSpec((BN, 1), lambda i: (i, 0)),
      ],
      out_specs=[
          pl.BlockSpec((G, F2), lambda i: (0, 0)),
          pl.BlockSpec((G, 1), lambda i: (0, 0)),
      ],
      out_shape=[
          jax.ShapeDtypeStruct((G, F2), jnp.float32),
          jax.ShapeDtypeStruct((G, 1), jnp.float32),
      ],
  )(h3, gate, m, bidx)


def _head_tc(sums, den, W_ih0, b_ih0, b_hh0, W_ih1, b_ih1, b_hh1,
             proj_W, proj_b):
  def cell(xin, W_ih, b_ih, b_hh):
    gi = lax.dot_general(xin, W_ih, (((1,), (1,)), ((), ())),
                         preferred_element_type=jnp.float32) + b_ih
    i_r, i_z, i_n = gi[:, :GH], gi[:, GH:2 * GH], gi[:, 2 * GH:]
    h_r, h_z, h_n = b_hh[:, :GH], b_hh[:, GH:2 * GH], b_hh[:, 2 * GH:]
    r = jax.nn.sigmoid(i_r + h_r)
    z = jax.nn.sigmoid(i_z + h_z)
    n = jnp.tanh(i_n + r * h_n)
    return (1.0 - z) * n

  def body(sums_ref, den_ref, wi0_ref, bi0_ref, bh0_ref,
           wi1_ref, bi1_ref, bh1_ref, pw_ref, pb_ref, out_ref):
    emb = sums_ref[...] / (den_ref[...] + 1e-16)
    g0 = cell(emb, wi0_ref[...], bi0_ref[...], bh0_ref[...])
    g1 = cell(g0, wi1_ref[...], bi1_ref[...], bh1_ref[...])
    out_ref[...] = jnp.dot(g1, pw_ref[...],
                           preferred_element_type=jnp.float32) + pb_ref[...]

  return pl.pallas_call(
      body,
      out_shape=jax.ShapeDtypeStruct((G, F2), jnp.float32),
  )(sums, den, W_ih0, b_ih0, b_hh0, W_ih1, b_ih1, b_hh1, proj_W, proj_b)


def kernel(x, edge_index, batch_idx, W0, b0, W1, b1, W2, b2, gate_W, gate_b,
           W_ih0, W_hh0, b_ih0, b_hh0, W_ih1, W_hh1, b_ih1, b_hh1,
           proj_W, proj_b):
  pad = jnp.stack([jnp.zeros((EPAD,), jnp.int32),
                   jnp.full((EPAD,), NP - 1, jnp.int32)])
  ep = jnp.concatenate([edge_index, pad], axis=1)

  deg2 = _deg_call(ep)
  us0, dinv = _pre0_tc(jnp.transpose(deg2), x, W0)

  S0 = _agg128(us0, ep)
  hs1 = _mid0_tc(S0, us0, dinv, b0[None, :])
  S1 = _agg128(hs1, ep)
  hs2a, hs2b = _gcn_tc([S1], [hs1], dinv, W1, b1[None, :], scale_out=True)
  S2a = _agg128(hs2a, ep)
  S2b = _agg128(hs2b, ep)
  (h3,) = _gcn_tc([S2a, S2b], [hs2a, hs2b], dinv, W2, b2[None, :],
                  scale_out=False)

  bidx = batch_idx[:, None]
  gate, m = _gate_tc(h3, gate_W, gate_b[None, :], bidx)
  sums, den = _pool_tc(h3, gate, m, bidx)
  return _head_tc(sums, den, W_ih0, b_ih0[None, :], b_hh0[None, :],
                  W_ih1, b_ih1[None, :], b_hh1[None, :], proj_W,
                  proj_b[None, :])
